# baseline (speedup 1.0000x reference)
import functools

import jax
import jax.numpy as jnp
from jax import lax
from jax.experimental import pallas as pl
from jax.experimental.pallas import tpu as pltpu
from jax.experimental.pallas import tpu_sc as plsc

N_NODES = 10000
N_EDGES = 160000
NW = 32
LANES = 16
E_PER_W = N_EDGES // NW
FULL_GROUPS = E_PER_W // LANES
TAIL = E_PER_W - FULL_GROUPS * LANES
EBUF = FULL_GROUPS * LANES + LANES
UNROLL = 4

NB = 8
NB_P = 10
BN = N_NODES // NB_P


def _make_edge_agg(feat):
    mesh = plsc.VectorSubcoreMesh(core_axis_name="c", subcore_axis_name="s")

    @functools.partial(
        pl.kernel,
        out_type=jax.ShapeDtypeStruct((NW * feat, N_NODES), jnp.float32),
        mesh=mesh,
        compiler_params=pltpu.CompilerParams(needs_layout_passes=False),
        scratch_types=[
            pltpu.VMEM((feat, N_NODES), jnp.float32),
            pltpu.VMEM((feat, N_NODES), jnp.float32),
            pltpu.VMEM((EBUF,), jnp.int32),
            pltpu.VMEM((EBUF,), jnp.int32),
            pltpu.SemaphoreType.DMA,
            pltpu.SemaphoreType.DMA,
            pltpu.SemaphoreType.DMA,
        ],
    )
    def edge_agg(p_hbm, ei_hbm, out_hbm, p_v, agg_v, src_v, dst_v,
                 sem_p, sem_s, sem_d):
        wid = lax.axis_index("s") * 2 + lax.axis_index("c")
        base = wid * E_PER_W
        cp_p = pltpu.async_copy(p_hbm, p_v, sem_p)
        cp_s = pltpu.async_copy(ei_hbm.at[pl.ds(base, E_PER_W)],
                                src_v.at[pl.ds(0, E_PER_W)], sem_s)
        cp_d = pltpu.async_copy(ei_hbm.at[pl.ds(N_EDGES + base, E_PER_W)],
                                dst_v.at[pl.ds(0, E_PER_W)], sem_d)

        zeros = jnp.zeros((LANES,), jnp.float32)

        def zero_body(i, carry):
            for u in range(5):
                for f in range(feat):
                    agg_v[f, pl.ds((i * 5 + u) * LANES, LANES)] = zeros
            return carry

        lax.fori_loop(0, N_NODES // LANES // 5, zero_body, 0)
        cp_p.wait()
        cp_s.wait()
        cp_d.wait()

        rows = [jnp.full((LANES,), f, jnp.int32) for f in range(feat)]

        def edge_body(i, carry):
            for u in range(UNROLL):
                e = (i * UNROLL + u) * LANES
                s = src_v[pl.ds(e, LANES)]
                d = dst_v[pl.ds(e, LANES)]
                for f in range(feat):
                    vals = plsc.load_gather(p_v, [rows[f], s])
                    plsc.addupdate_scatter(agg_v, [rows[f], d], vals)
            return carry

        lax.fori_loop(0, FULL_GROUPS // UNROLL, edge_body, 0)

        mask = lax.iota(jnp.int32, LANES) < TAIL
        e = FULL_GROUPS * LANES
        s = jnp.where(mask, src_v[pl.ds(e, LANES)], 0)
        d = jnp.where(mask, dst_v[pl.ds(e, LANES)], 0)
        for f in range(feat):
            vals = plsc.load_gather(p_v, [rows[f], s])
            plsc.addupdate_scatter(agg_v, [rows[f], d], vals, mask=mask)

        pltpu.sync_copy(agg_v, out_hbm.at[pl.ds(wid * feat, feat)])

    return edge_agg


_edge_agg_f4 = _make_edge_agg(4)
_edge_agg_f2 = _make_edge_agg(2)


def _bias_rows(b, shape):
    return lax.broadcast_in_dim(b, shape, (0,))


_CHUNK = 1024
_CHUNKS = [(o, min(_CHUNK, N_NODES - o)) for o in range(0, N_NODES, _CHUNK)]


def _proj_kernel(x_hbm, wr_ref, wo_ref, b_ref, p_ref, r_ref,
                 xb0, xb1, sem0, sem1):
    bufs, sems = (xb0, xb1), (sem0, sem1)
    copies = []
    for ci, (off, sz) in enumerate(_CHUNKS):
        copies.append(pltpu.make_async_copy(
            x_hbm.at[pl.ds(off, sz), :], bufs[ci % 2].at[pl.ds(0, sz), :],
            sems[ci % 2]))
    copies[0].start()
    copies[1].start()
    for ci, (off, sz) in enumerate(_CHUNKS):
        copies[ci].wait()
        xb = bufs[ci % 2][pl.ds(0, sz), :]
        p_ref[:, pl.ds(off, sz)] = lax.dot_general(
            wr_ref[...], xb, (((0,), (1,)), ((), ())),
            preferred_element_type=jnp.float32)
        r = lax.dot_general(wo_ref[...], xb, (((0,), (1,)), ((), ())),
                            preferred_element_type=jnp.float32)
        r_ref[:, pl.ds(off, sz)] = r + _bias_rows(b_ref[...], r.shape)
        if ci + 2 < len(_CHUNKS):
            copies[ci + 2].start()


def _project(x, w_rel, w_root, b):
    fr, fo = w_rel.shape[1], w_root.shape[1]
    d = x.shape[1]
    return pl.pallas_call(
        _proj_kernel,
        in_specs=[
            pl.BlockSpec(memory_space=pl.ANY),
            pl.BlockSpec((d, fr), lambda: (0, 0)),
            pl.BlockSpec((d, fo), lambda: (0, 0)),
            pl.BlockSpec((fo,), lambda: (0,)),
        ],
        out_specs=(
            pl.BlockSpec((fr, N_NODES), lambda: (0, 0)),
            pl.BlockSpec((fo, N_NODES), lambda: (0, 0)),
        ),
        out_shape=(
            jax.ShapeDtypeStruct((fr, N_NODES), jnp.float32),
            jax.ShapeDtypeStruct((fo, N_NODES), jnp.float32),
        ),
        scratch_shapes=[
            pltpu.VMEM((_CHUNK, d), jnp.float32),
            pltpu.VMEM((_CHUNK, d), jnp.float32),
            pltpu.SemaphoreType.DMA,
            pltpu.SemaphoreType.DMA,
        ],
    )(x, w_rel, w_root, b)


def _combine_proj_kernel(parts_ref, r_ref, wr_ref, wo_ref, b_ref,
                         p_ref, rn_ref, acc_ref):
    i = pl.program_id(0)
    feat = r_ref.shape[0]
    chunk = parts_ref[...].reshape(-1, feat, N_NODES)
    s = jnp.sum(chunk, axis=0)

    @pl.when(i == 0)
    def _init():
        acc_ref[...] = s

    @pl.when(i > 0)
    def _accum():
        acc_ref[...] += s

    @pl.when(i == NB - 1)
    def _final():
        h = jnp.tanh(acc_ref[...] + r_ref[...])
        p_ref[...] = lax.dot_general(wr_ref[...], h, (((0,), (0,)), ((), ())),
                                     preferred_element_type=jnp.float32)
        rn = lax.dot_general(wo_ref[...], h, (((0,), (0,)), ((), ())),
                             preferred_element_type=jnp.float32)
        rn_ref[...] = rn + _bias_rows(b_ref[...], rn.shape)


def _combine_project(partials, r, w_rel, w_root, b):
    feat = r.shape[0]
    fr, fo = w_rel.shape[1], w_root.shape[1]
    rows = NW * feat // NB
    return pl.pallas_call(
        _combine_proj_kernel,
        grid=(NB,),
        in_specs=[
            pl.BlockSpec((rows, N_NODES), lambda i: (i, 0)),
            pl.BlockSpec((feat, N_NODES), lambda i: (0, 0)),
            pl.BlockSpec((feat, fr), lambda i: (0, 0)),
            pl.BlockSpec((feat, fo), lambda i: (0, 0)),
            pl.BlockSpec((fo,), lambda i: (0,)),
        ],
        out_specs=(
            pl.BlockSpec((fr, N_NODES), lambda i: (0, 0)),
            pl.BlockSpec((fo, N_NODES), lambda i: (0, 0)),
        ),
        out_shape=(
            jax.ShapeDtypeStruct((fr, N_NODES), jnp.float32),
            jax.ShapeDtypeStruct((fo, N_NODES), jnp.float32),
        ),
        scratch_shapes=[pltpu.VMEM((feat, N_NODES), jnp.float32)],
    )(partials, r, w_rel, w_root, b)


def _finish_kernel(parts_ref, r_ref, out_ref, acc_ref):
    i = pl.program_id(0)
    feat = r_ref.shape[0]
    chunk = parts_ref[...].reshape(-1, feat, N_NODES)
    s = jnp.sum(chunk, axis=0)

    @pl.when(i == 0)
    def _init():
        acc_ref[...] = s

    @pl.when(i > 0)
    def _accum():
        acc_ref[...] += s

    @pl.when(i == NB - 1)
    def _final():
        out_ref[...] = jnp.tanh(acc_ref[...] + r_ref[...])


def _finish(partials, r):
    feat = r.shape[0]
    rows = NW * feat // NB
    return pl.pallas_call(
        _finish_kernel,
        grid=(NB,),
        in_specs=[
            pl.BlockSpec((rows, N_NODES), lambda i: (i, 0)),
            pl.BlockSpec((feat, N_NODES), lambda i: (0, 0)),
        ],
        out_specs=pl.BlockSpec((feat, N_NODES), lambda i: (0, 0)),
        out_shape=jax.ShapeDtypeStruct(r.shape, jnp.float32),
        scratch_shapes=[pltpu.VMEM((feat, N_NODES), jnp.float32)],
    )(partials, r)


def kernel(edge_index, x, W1_rel, b1_rel, W1_root, W2_rel, b2_rel, W2_root,
           W3_rel, b3_rel, W3_root):
    ei_flat = edge_index.reshape(-1)
    p1, r1 = _project(x, W1_rel, W1_root, b1_rel)
    parts1 = _edge_agg_f4(p1, ei_flat)
    p2, r2 = _combine_project(parts1, r1, W2_rel, W2_root, b2_rel)
    parts2 = _edge_agg_f4(p2, ei_flat)
    p3, r3 = _combine_project(parts2, r2, W3_rel, W3_root, b3_rel)
    parts3 = _edge_agg_f2(p3, ei_flat)
    return _finish(parts3, r3).T


# speedup vs baseline: 1.0993x; 1.0993x over previous
import functools

import jax
import jax.numpy as jnp
from jax import lax
from jax.experimental import pallas as pl
from jax.experimental.pallas import tpu as pltpu
from jax.experimental.pallas import tpu_sc as plsc

N_NODES = 10000
N_EDGES = 160000
NW = 32
LANES = 16
E_PER_W = N_EDGES // NW
FULL_GROUPS = E_PER_W // LANES
TAIL = E_PER_W - FULL_GROUPS * LANES
EBUF = FULL_GROUPS * LANES + LANES
UNROLL = 4

NB = 8
NB_P = 10
BN = N_NODES // NB_P


def _make_edge_agg(feat):
    mesh = plsc.VectorSubcoreMesh(core_axis_name="c", subcore_axis_name="s")

    @functools.partial(
        pl.kernel,
        out_type=jax.ShapeDtypeStruct((NW * feat, N_NODES), jnp.float32),
        mesh=mesh,
        compiler_params=pltpu.CompilerParams(needs_layout_passes=False),
        scratch_types=[
            pltpu.VMEM((feat, N_NODES), jnp.float32),
            pltpu.VMEM((feat, N_NODES), jnp.float32),
            pltpu.VMEM((EBUF,), jnp.int32),
            pltpu.VMEM((EBUF,), jnp.int32),
            pltpu.SemaphoreType.DMA,
            pltpu.SemaphoreType.DMA,
            pltpu.SemaphoreType.DMA,
        ],
    )
    def edge_agg(p_hbm, ei_hbm, out_hbm, p_v, agg_v, src_v, dst_v,
                 sem_p, sem_s, sem_d):
        wid = lax.axis_index("s") * 2 + lax.axis_index("c")
        base = wid * E_PER_W
        cp_p = pltpu.async_copy(p_hbm, p_v, sem_p)
        cp_s = pltpu.async_copy(ei_hbm.at[pl.ds(base, E_PER_W)],
                                src_v.at[pl.ds(0, E_PER_W)], sem_s)
        cp_d = pltpu.async_copy(ei_hbm.at[pl.ds(N_EDGES + base, E_PER_W)],
                                dst_v.at[pl.ds(0, E_PER_W)], sem_d)

        zeros = jnp.zeros((LANES,), jnp.float32)

        def zero_body(i, carry):
            for u in range(5):
                for f in range(feat):
                    agg_v[f, pl.ds((i * 5 + u) * LANES, LANES)] = zeros
            return carry

        lax.fori_loop(0, N_NODES // LANES // 5, zero_body, 0)
        cp_p.wait()
        cp_s.wait()
        cp_d.wait()

        rows = [jnp.full((LANES,), f, jnp.int32) for f in range(feat)]

        def edge_body(i, carry):
            for u in range(UNROLL):
                e = (i * UNROLL + u) * LANES
                s = src_v[pl.ds(e, LANES)]
                d = dst_v[pl.ds(e, LANES)]
                for f in range(feat):
                    vals = plsc.load_gather(p_v, [rows[f], s])
                    plsc.addupdate_scatter(agg_v, [rows[f], d], vals)
            return carry

        lax.fori_loop(0, FULL_GROUPS // UNROLL, edge_body, 0)

        mask = lax.iota(jnp.int32, LANES) < TAIL
        e = FULL_GROUPS * LANES
        s = jnp.where(mask, src_v[pl.ds(e, LANES)], 0)
        d = jnp.where(mask, dst_v[pl.ds(e, LANES)], 0)
        for f in range(feat):
            vals = plsc.load_gather(p_v, [rows[f], s])
            plsc.addupdate_scatter(agg_v, [rows[f], d], vals, mask=mask)

        pltpu.sync_copy(agg_v, out_hbm.at[pl.ds(wid * feat, feat)])

    return edge_agg


_edge_agg_f4 = _make_edge_agg(4)
_edge_agg_f2 = _make_edge_agg(2)


def _bias_rows(b, shape):
    return lax.broadcast_in_dim(b, shape, (0,))


def _proj_kernel(x_ref, wr_ref, wo_ref, b_ref, p_ref, r_ref):
    x = x_ref[...]
    p_ref[...] = lax.dot_general(wr_ref[...], x, (((0,), (1,)), ((), ())),
                                 preferred_element_type=jnp.float32)
    r = lax.dot_general(wo_ref[...], x, (((0,), (1,)), ((), ())),
                        preferred_element_type=jnp.float32)
    r_ref[...] = r + _bias_rows(b_ref[...], r.shape)


def _project(x, w_rel, w_root, b):
    return pl.pallas_call(
        _proj_kernel,
        out_shape=(
            jax.ShapeDtypeStruct((w_rel.shape[1], N_NODES), jnp.float32),
            jax.ShapeDtypeStruct((w_root.shape[1], N_NODES), jnp.float32),
        ),
    )(x, w_rel, w_root, b)


def _combine_proj_kernel(parts_ref, r_ref, wr_ref, wo_ref, b_ref,
                         p_ref, rn_ref):
    feat = r_ref.shape[0]
    parts = parts_ref[...].reshape(NW, feat, N_NODES)
    h = jnp.tanh(jnp.sum(parts, axis=0) + r_ref[...])
    p_ref[...] = lax.dot_general(wr_ref[...], h, (((0,), (0,)), ((), ())),
                                 preferred_element_type=jnp.float32)
    rn = lax.dot_general(wo_ref[...], h, (((0,), (0,)), ((), ())),
                         preferred_element_type=jnp.float32)
    rn_ref[...] = rn + _bias_rows(b_ref[...], rn.shape)


def _combine_project(partials, r, w_rel, w_root, b):
    return pl.pallas_call(
        _combine_proj_kernel,
        out_shape=(
            jax.ShapeDtypeStruct((w_rel.shape[1], N_NODES), jnp.float32),
            jax.ShapeDtypeStruct((w_root.shape[1], N_NODES), jnp.float32),
        ),
    )(partials, r, w_rel, w_root, b)


def _finish_kernel(parts_ref, r_ref, out_ref):
    feat = r_ref.shape[0]
    parts = parts_ref[...].reshape(NW, feat, N_NODES)
    out_ref[...] = jnp.tanh(jnp.sum(parts, axis=0) + r_ref[...])


def _finish(partials, r):
    return pl.pallas_call(
        _finish_kernel,
        out_shape=jax.ShapeDtypeStruct(r.shape, jnp.float32),
    )(partials, r)


def kernel(edge_index, x, W1_rel, b1_rel, W1_root, W2_rel, b2_rel, W2_root,
           W3_rel, b3_rel, W3_root):
    ei_flat = edge_index.reshape(-1)
    p1, r1 = _project(x, W1_rel, W1_root, b1_rel)
    parts1 = _edge_agg_f4(p1, ei_flat)
    p2, r2 = _combine_project(parts1, r1, W2_rel, W2_root, b2_rel)
    parts2 = _edge_agg_f4(p2, ei_flat)
    p3, r3 = _combine_project(parts2, r2, W3_rel, W3_root, b3_rel)
    parts3 = _edge_agg_f2(p3, ei_flat)
    return _finish(parts3, r3).T
